# baseline (speedup 1.0000x reference)
import functools

import jax
import jax.numpy as jnp
from jax import lax
from jax.experimental import pallas as pl
from jax.experimental.pallas import tpu as pltpu
from jax.experimental.pallas import tpu_sc as plsc

N = 10000
E = 320000
D = 128
DOUT = 16
EPS = 1e-5

NCORES = 2
NSUB = 16
NWORK = NCORES * NSUB
CHUNK = 128
TCHUNK = E // CHUNK
NCHUNK = TCHUNK // NWORK
NEXTRA = TCHUNK - NCHUNK * NWORK
EPT = NCHUNK * CHUNK
NPAD = 10240
STRIPE = NPAD // NSUB
NBUF = 2
HALF = NCHUNK // 2
HALFE = HALF * CHUNK

BS = 2000
GRID = N // BS

_sc_mesh = plsc.VectorSubcoreMesh(core_axis_name="c", subcore_axis_name="s")


def _make_deg_kernel():
    @functools.partial(
        pl.kernel,
        out_type=jax.ShapeDtypeStruct((NCORES, NPAD), jnp.float32),
        mesh=_sc_mesh,
        scratch_types=(
            [
                pltpu.VMEM((2, NCHUNK * CHUNK), jnp.int32),
                pltpu.VMEM((CHUNK,), jnp.float32),
            ]
            + [pltpu.VMEM((CHUNK,), jnp.int32) for _ in range(13)]
            + [
                pltpu.VMEM((2, CHUNK), jnp.int32),
                pltpu.VMEM_SHARED((NPAD,), jnp.float32),
                pltpu.SemaphoreType.DMA,
                pltpu.SemaphoreType.DMA,
            ]
        ),
    )
    def deg_kernel(ei_hbm, out_hbm, eall_v, ones_v, *rest):
        dcur = rest[:13]
        ex_v, acc_sh, isem, ssem = rest[13:]
        c = lax.axis_index("c")
        s = lax.axis_index("s")
        wid = c * NSUB + s

        icopy = pltpu.async_copy(
            ei_hbm.at[:, pl.ds(wid * EPT, NCHUNK * CHUNK)], eall_v, isem)

        @pl.loop(0, CHUNK // 16)
        def _(j):
            ones_v[pl.ds(j * 16, 16)] = jnp.zeros((16,), jnp.float32)

        for k in range(STRIPE // CHUNK):
            pltpu.sync_copy(ones_v, acc_sh.at[pl.ds(s * STRIPE + k * CHUNK, CHUNK)])

        @pl.loop(0, CHUNK // 16)
        def _(j):
            ones_v[pl.ds(j * 16, 16)] = jnp.ones((16,), jnp.float32)

        icopy.wait()
        plsc.subcore_barrier()

        @pl.loop(0, NCHUNK, step=13)
        def _(i0):
            for b in range(13):
                i = i0 + b
                for j in range(CHUNK // 16):
                    dcur[b][pl.ds(j * 16, 16)] = (
                        eall_v[1, pl.ds(i * CHUNK + j * 16, 16)])
                pltpu.async_copy(ones_v, acc_sh.at[dcur[b]], ssem, add=True)
            for b in range(13):
                pltpu.make_async_copy(ones_v, acc_sh.at[dcur[0]], ssem).wait()

        @pl.when(wid < NEXTRA)
        def _():
            pltpu.sync_copy(
                ei_hbm.at[:, pl.ds((NCHUNK * NWORK + wid) * CHUNK, CHUNK)],
                ex_v)
            for j in range(CHUNK // 16):
                dcur[0][pl.ds(j * 16, 16)] = ex_v[1, pl.ds(j * 16, 16)]
            pltpu.sync_copy(ones_v, acc_sh.at[dcur[0]], add=True)

        plsc.subcore_barrier()
        pltpu.sync_copy(acc_sh.at[pl.ds(s * STRIPE, STRIPE)],
                        out_hbm.at[c, pl.ds(s * STRIPE, STRIPE)])

    return deg_kernel


def _make_agg_kernel(d):
    @functools.partial(
        pl.kernel,
        out_type=jax.ShapeDtypeStruct((NCORES, NPAD, d), jnp.float32),
        mesh=_sc_mesh,
        scratch_types=(
            [pltpu.VMEM((2, HALFE), jnp.int32)]
            + [pltpu.VMEM((CHUNK, d), jnp.float32) for _ in range(NBUF)]
            + [pltpu.VMEM((CHUNK,), jnp.int32) for _ in range(NBUF)]
            + [pltpu.VMEM((2, CHUNK), jnp.int32)]
            + [pltpu.VMEM_SHARED((NPAD, d), jnp.float32)]
            + [pltpu.SemaphoreType.DMA for _ in range(2 * NBUF + 1)]
        ),
    )
    def agg_kernel(hs_hbm, ei_hbm, out_hbm, eall_v,
                   r0, r1, d0, d1, ex_v, acc_sh, g0, g1, s0, s1, isem):
        rows = (r0, r1)
        dcur = (d0, d1)
        gsem = (g0, g1)
        ssem = (s0, s1)
        c = lax.axis_index("c")
        s = lax.axis_index("s")
        wid = c * NSUB + s

        ic1 = pltpu.async_copy(
            ei_hbm.at[:, pl.ds(wid * EPT, HALFE)], eall_v, isem)

        @pl.loop(0, CHUNK)
        def _(i):
            for j in range(d // 16):
                r0[i, pl.ds(j * 16, 16)] = jnp.zeros((16,), jnp.float32)

        for k in range(STRIPE // CHUNK):
            pltpu.async_copy(
                r0, acc_sh.at[pl.ds(s * STRIPE + k * CHUNK, CHUNK)], g0)
        for k in range(STRIPE // CHUNK):
            pltpu.make_async_copy(
                r0, acc_sh.at[pl.ds(s * STRIPE, CHUNK)], g0).wait()

        ic1.wait()
        plsc.subcore_barrier()

        for h in range(NCHUNK // HALF):
            if h > 0:
                pltpu.sync_copy(
                    ei_hbm.at[:, pl.ds(wid * EPT + h * HALFE, HALFE)],
                    eall_v)

            @pl.loop(0, HALF + NBUF, step=NBUF)
            def _(i0):
                for b in range(NBUF):
                    i = i0 + b

                    @pl.when(i < HALF)
                    def _():
                        @pl.when(i >= NBUF)
                        def _():
                            pltpu.make_async_copy(
                                rows[b], acc_sh.at[dcur[b]], ssem[b]).wait()
                        pltpu.async_copy(
                            hs_hbm.at[eall_v.at[0, pl.ds(i * CHUNK, CHUNK)]],
                            rows[b], gsem[b])

                    cc = i - 1
                    bb = (b - 1) % NBUF

                    @pl.when((cc >= 0) & (cc < HALF))
                    def _():
                        pltpu.make_async_copy(
                            hs_hbm.at[eall_v.at[0, pl.ds(cc * CHUNK, CHUNK)]],
                            rows[bb], gsem[bb]).wait()
                        for j in range(CHUNK // 16):
                            dcur[bb][pl.ds(j * 16, 16)] = (
                                eall_v[1, pl.ds(cc * CHUNK + j * 16, 16)])
                        pltpu.async_copy(rows[bb], acc_sh.at[dcur[bb]],
                                         ssem[bb], add=True)

            for b in range(NBUF):
                pltpu.make_async_copy(rows[b], acc_sh.at[dcur[b]],
                                      ssem[b]).wait()

        @pl.when(wid < NEXTRA)
        def _():
            pltpu.sync_copy(
                ei_hbm.at[:, pl.ds((NCHUNK * NWORK + wid) * CHUNK, CHUNK)],
                ex_v)
            pltpu.sync_copy(hs_hbm.at[ex_v.at[0]], rows[0])
            for j in range(CHUNK // 16):
                dcur[0][pl.ds(j * 16, 16)] = ex_v[1, pl.ds(j * 16, 16)]
            pltpu.sync_copy(rows[0], acc_sh.at[dcur[0]], add=True)

        plsc.subcore_barrier()
        pltpu.sync_copy(acc_sh.at[pl.ds(s * STRIPE, STRIPE)],
                        out_hbm.at[c, pl.ds(s * STRIPE, STRIPE)])

    return agg_kernel


_deg_call = _make_deg_kernel()
_agg128 = _make_agg_kernel(D)


def _dinv_of(degT_ref):
    deg = degT_ref[:, 0:1] + degT_ref[:, 1:2] + 1.0
    return lax.rsqrt(deg)


def _tc1_body(x_ref, w_ref, degT_ref, hs_ref):
    t = jnp.dot(x_ref[...], w_ref[...], preferred_element_type=jnp.float32)
    hs_ref[...] = t * _dinv_of(degT_ref)


def _tc1(x, W1, degT):
    return pl.pallas_call(
        _tc1_body,
        grid=(GRID,),
        in_specs=[
            pl.BlockSpec((BS, D), lambda i: (i, 0)),
            pl.BlockSpec((D, D), lambda i: (0, 0)),
            pl.BlockSpec((BS, 2), lambda i: (i, 0)),
        ],
        out_specs=pl.BlockSpec((BS, D), lambda i: (i, 0)),
        out_shape=jax.ShapeDtypeStruct((N, D), jnp.float32),
    )(x, W1, degT)


def _tcmid_body(parts_ref, hs_ref, degT_ref, b_ref, g_ref, bb_ref, w_ref, out_ref):
    p = parts_ref[...]
    dinv = _dinv_of(degT_ref)
    y = dinv * (p[0] + p[1] + hs_ref[...]) + b_ref[...]
    mean = jnp.mean(y, axis=1, keepdims=True)
    yc = y - mean
    var = jnp.mean(yc * yc, axis=1, keepdims=True)
    yn = yc * lax.rsqrt(var + EPS) * g_ref[...] + bb_ref[...]
    r = jnp.maximum(yn, 0.0)
    t = jnp.dot(r, w_ref[...], preferred_element_type=jnp.float32)
    out_ref[...] = t * dinv


def _tcmid(parts, hs, degT, b, g, bb, W):
    return pl.pallas_call(
        _tcmid_body,
        grid=(GRID,),
        in_specs=[
            pl.BlockSpec((NCORES, BS, D), lambda i: (0, i, 0)),
            pl.BlockSpec((BS, D), lambda i: (i, 0)),
            pl.BlockSpec((BS, 2), lambda i: (i, 0)),
            pl.BlockSpec((1, D), lambda i: (0, 0)),
            pl.BlockSpec((1, D), lambda i: (0, 0)),
            pl.BlockSpec((1, D), lambda i: (0, 0)),
            pl.BlockSpec((D, D), lambda i: (0, 0)),
        ],
        out_specs=pl.BlockSpec((BS, D), lambda i: (i, 0)),
        out_shape=jax.ShapeDtypeStruct((N, D), jnp.float32),
    )(parts, hs, degT, b, g, bb, W)


def _tclast_body(parts_ref, hs_ref, degT_ref, b_ref, out_ref):
    p = parts_ref[...]
    dinv = _dinv_of(degT_ref)
    y = dinv * (p[0, :, :DOUT] + p[1, :, :DOUT] + hs_ref[...][:, :DOUT]) + b_ref[...]
    m = jnp.max(y, axis=1, keepdims=True)
    ym = y - m
    out_ref[...] = ym - jnp.log(jnp.sum(jnp.exp(ym), axis=1, keepdims=True))


def _tclast(parts, hs, degT, b):
    return pl.pallas_call(
        _tclast_body,
        grid=(GRID,),
        in_specs=[
            pl.BlockSpec((NCORES, BS, D), lambda i: (0, i, 0)),
            pl.BlockSpec((BS, D), lambda i: (i, 0)),
            pl.BlockSpec((BS, 2), lambda i: (i, 0)),
            pl.BlockSpec((1, DOUT), lambda i: (0, 0)),
        ],
        out_specs=pl.BlockSpec((BS, DOUT), lambda i: (i, 0)),
        out_shape=jax.ShapeDtypeStruct((N, DOUT), jnp.float32),
    )(parts, hs, degT, b)


def kernel(x, edge_index, W1, b1, W2, b2, W3, b3, ln1_g, ln1_b, ln2_g, ln2_b):
    ei_pad = edge_index

    parts_deg = _deg_call(ei_pad)
    degT = parts_deg[:, :N].T

    hs1 = _tc1(x, W1, degT)
    parts1 = _agg128(hs1, ei_pad)
    hs2 = _tcmid(parts1, hs1, degT, b1.reshape(1, D), ln1_g.reshape(1, D),
                 ln1_b.reshape(1, D), W2)
    parts2 = _agg128(hs2, ei_pad)
    W3p = jnp.pad(W3, ((0, 0), (0, D - DOUT)))
    hs3 = _tcmid(parts2, hs2, degT, b2.reshape(1, D), ln2_g.reshape(1, D),
                 ln2_b.reshape(1, D), W3p)
    parts3 = _agg128(hs3, ei_pad)
    return _tclast(parts3, hs3, degT, b3.reshape(1, DOUT))


# speedup vs baseline: 1.0018x; 1.0018x over previous
import functools

import jax
import jax.numpy as jnp
from jax import lax
from jax.experimental import pallas as pl
from jax.experimental.pallas import tpu as pltpu
from jax.experimental.pallas import tpu_sc as plsc

N = 10000
E = 320000
D = 128
DOUT = 16
EPS = 1e-5

NCORES = 2
NSUB = 16
NWORK = NCORES * NSUB
CHUNK = 128
TCHUNK = E // CHUNK
NCHUNK = TCHUNK // NWORK
NEXTRA = TCHUNK - NCHUNK * NWORK
EPT = NCHUNK * CHUNK
NPAD = 10240
STRIPE = NPAD // NSUB
NBUF = 2
HALF = NCHUNK // 2
HALFE = HALF * CHUNK

BS = 2000
GRID = N // BS

_BUILT = {}


def _sc_mesh():
    return plsc.VectorSubcoreMesh(core_axis_name="c", subcore_axis_name="s",
                                  num_cores=NCORES, num_subcores=NSUB)


def _built(name, builder, *args):
    key = (name,) + args
    if key not in _BUILT:
        _BUILT[key] = builder(*args)
    return _BUILT[key]


def _make_deg_kernel():
    @functools.partial(
        pl.kernel,
        out_type=jax.ShapeDtypeStruct((NCORES, NPAD), jnp.float32),
        mesh=_sc_mesh(),
        scratch_types=(
            [
                pltpu.VMEM((2, NCHUNK * CHUNK), jnp.int32),
                pltpu.VMEM((CHUNK,), jnp.float32),
            ]
            + [pltpu.VMEM((CHUNK,), jnp.int32) for _ in range(13)]
            + [
                pltpu.VMEM((2, CHUNK), jnp.int32),
                pltpu.VMEM_SHARED((NPAD,), jnp.float32),
                pltpu.SemaphoreType.DMA,
                pltpu.SemaphoreType.DMA,
            ]
        ),
    )
    def deg_kernel(ei_hbm, out_hbm, eall_v, ones_v, *rest):
        dcur = rest[:13]
        ex_v, acc_sh, isem, ssem = rest[13:]
        c = lax.axis_index("c")
        s = lax.axis_index("s")
        wid = c * NSUB + s

        icopy = pltpu.async_copy(
            ei_hbm.at[:, pl.ds(wid * EPT, NCHUNK * CHUNK)], eall_v, isem)

        @pl.loop(0, CHUNK // 16)
        def _(j):
            ones_v[pl.ds(j * 16, 16)] = jnp.zeros((16,), jnp.float32)

        for k in range(STRIPE // CHUNK):
            pltpu.sync_copy(ones_v, acc_sh.at[pl.ds(s * STRIPE + k * CHUNK, CHUNK)])

        @pl.loop(0, CHUNK // 16)
        def _(j):
            ones_v[pl.ds(j * 16, 16)] = jnp.ones((16,), jnp.float32)

        icopy.wait()
        plsc.subcore_barrier()

        @pl.loop(0, NCHUNK, step=13)
        def _(i0):
            for b in range(13):
                i = i0 + b
                for j in range(CHUNK // 16):
                    dcur[b][pl.ds(j * 16, 16)] = (
                        eall_v[1, pl.ds(i * CHUNK + j * 16, 16)])
                pltpu.async_copy(ones_v, acc_sh.at[dcur[b]], ssem, add=True)
            for b in range(13):
                pltpu.make_async_copy(ones_v, acc_sh.at[dcur[0]], ssem).wait()

        @pl.when(wid < NEXTRA)
        def _():
            pltpu.sync_copy(
                ei_hbm.at[:, pl.ds((NCHUNK * NWORK + wid) * CHUNK, CHUNK)],
                ex_v)
            for j in range(CHUNK // 16):
                dcur[0][pl.ds(j * 16, 16)] = ex_v[1, pl.ds(j * 16, 16)]
            pltpu.sync_copy(ones_v, acc_sh.at[dcur[0]], add=True)

        plsc.subcore_barrier()
        pltpu.sync_copy(acc_sh.at[pl.ds(s * STRIPE, STRIPE)],
                        out_hbm.at[c, pl.ds(s * STRIPE, STRIPE)])

    return deg_kernel


def _make_agg_kernel(d):
    @functools.partial(
        pl.kernel,
        out_type=jax.ShapeDtypeStruct((NCORES, NPAD, d), jnp.float32),
        mesh=_sc_mesh(),
        scratch_types=(
            [pltpu.VMEM((2, HALFE), jnp.int32)]
            + [pltpu.VMEM((CHUNK, d), jnp.float32) for _ in range(NBUF)]
            + [pltpu.VMEM((CHUNK,), jnp.int32) for _ in range(NBUF)]
            + [pltpu.VMEM((2, CHUNK), jnp.int32)]
            + [pltpu.VMEM_SHARED((NPAD, d), jnp.float32)]
            + [pltpu.SemaphoreType.DMA for _ in range(2 * NBUF + 1)]
        ),
    )
    def agg_kernel(hs_hbm, ei_hbm, out_hbm, eall_v,
                   r0, r1, d0, d1, ex_v, acc_sh, g0, g1, s0, s1, isem):
        rows = (r0, r1)
        dcur = (d0, d1)
        gsem = (g0, g1)
        ssem = (s0, s1)
        c = lax.axis_index("c")
        s = lax.axis_index("s")
        wid = c * NSUB + s

        ic1 = pltpu.async_copy(
            ei_hbm.at[:, pl.ds(wid * EPT, HALFE)], eall_v, isem)

        @pl.loop(0, CHUNK)
        def _(i):
            for j in range(d // 16):
                r0[i, pl.ds(j * 16, 16)] = jnp.zeros((16,), jnp.float32)

        for k in range(STRIPE // CHUNK):
            pltpu.async_copy(
                r0, acc_sh.at[pl.ds(s * STRIPE + k * CHUNK, CHUNK)], g0)
        for k in range(STRIPE // CHUNK):
            pltpu.make_async_copy(
                r0, acc_sh.at[pl.ds(s * STRIPE, CHUNK)], g0).wait()

        ic1.wait()
        plsc.subcore_barrier()

        for h in range(NCHUNK // HALF):
            if h > 0:
                pltpu.sync_copy(
                    ei_hbm.at[:, pl.ds(wid * EPT + h * HALFE, HALFE)],
                    eall_v)

            @pl.loop(0, HALF + NBUF, step=NBUF)
            def _(i0):
                for b in range(NBUF):
                    i = i0 + b

                    @pl.when(i < HALF)
                    def _():
                        @pl.when(i >= NBUF)
                        def _():
                            pltpu.make_async_copy(
                                rows[b], acc_sh.at[dcur[b]], ssem[b]).wait()
                        pltpu.async_copy(
                            hs_hbm.at[eall_v.at[0, pl.ds(i * CHUNK, CHUNK)]],
                            rows[b], gsem[b])

                    cc = i - 1
                    bb = (b - 1) % NBUF

                    @pl.when((cc >= 0) & (cc < HALF))
                    def _():
                        pltpu.make_async_copy(
                            hs_hbm.at[eall_v.at[0, pl.ds(cc * CHUNK, CHUNK)]],
                            rows[bb], gsem[bb]).wait()
                        for j in range(CHUNK // 16):
                            dcur[bb][pl.ds(j * 16, 16)] = (
                                eall_v[1, pl.ds(cc * CHUNK + j * 16, 16)])
                        pltpu.async_copy(rows[bb], acc_sh.at[dcur[bb]],
                                         ssem[bb], add=True)

            for b in range(NBUF):
                pltpu.make_async_copy(rows[b], acc_sh.at[dcur[b]],
                                      ssem[b]).wait()

        @pl.when(wid < NEXTRA)
        def _():
            pltpu.sync_copy(
                ei_hbm.at[:, pl.ds((NCHUNK * NWORK + wid) * CHUNK, CHUNK)],
                ex_v)
            pltpu.sync_copy(hs_hbm.at[ex_v.at[0]], rows[0])
            for j in range(CHUNK // 16):
                dcur[0][pl.ds(j * 16, 16)] = ex_v[1, pl.ds(j * 16, 16)]
            pltpu.sync_copy(rows[0], acc_sh.at[dcur[0]], add=True)

        plsc.subcore_barrier()
        pltpu.sync_copy(acc_sh.at[pl.ds(s * STRIPE, STRIPE)],
                        out_hbm.at[c, pl.ds(s * STRIPE, STRIPE)])

    return agg_kernel


def _deg_call(ei):
    return _built("deg", _make_deg_kernel)(ei)


def _agg128(hs, ei):
    return _built("agg", _make_agg_kernel, D)(hs, ei)


def _dinv_of(degT_ref):
    deg = degT_ref[:, 0:1] + degT_ref[:, 1:2] + 1.0
    return lax.rsqrt(deg)


def _tc1_body(x_ref, w_ref, degT_ref, hs_ref):
    t = jnp.dot(x_ref[...], w_ref[...], preferred_element_type=jnp.float32)
    hs_ref[...] = t * _dinv_of(degT_ref)


def _tc1(x, W1, degT):
    return pl.pallas_call(
        _tc1_body,
        grid=(GRID,),
        in_specs=[
            pl.BlockSpec((BS, D), lambda i: (i, 0)),
            pl.BlockSpec((D, D), lambda i: (0, 0)),
            pl.BlockSpec((BS, 2), lambda i: (i, 0)),
        ],
        out_specs=pl.BlockSpec((BS, D), lambda i: (i, 0)),
        out_shape=jax.ShapeDtypeStruct((N, D), jnp.float32),
    )(x, W1, degT)


def _tcmid_body(parts_ref, hs_ref, degT_ref, b_ref, g_ref, bb_ref, w_ref, out_ref):
    p = parts_ref[...]
    dinv = _dinv_of(degT_ref)
    y = dinv * (p[0] + p[1] + hs_ref[...]) + b_ref[...]
    mean = jnp.mean(y, axis=1, keepdims=True)
    yc = y - mean
    var = jnp.mean(yc * yc, axis=1, keepdims=True)
    yn = yc * lax.rsqrt(var + EPS) * g_ref[...] + bb_ref[...]
    r = jnp.maximum(yn, 0.0)
    t = jnp.dot(r, w_ref[...], preferred_element_type=jnp.float32)
    out_ref[...] = t * dinv


def _tcmid(parts, hs, degT, b, g, bb, W):
    return pl.pallas_call(
        _tcmid_body,
        grid=(GRID,),
        in_specs=[
            pl.BlockSpec((NCORES, BS, D), lambda i: (0, i, 0)),
            pl.BlockSpec((BS, D), lambda i: (i, 0)),
            pl.BlockSpec((BS, 2), lambda i: (i, 0)),
            pl.BlockSpec((1, D), lambda i: (0, 0)),
            pl.BlockSpec((1, D), lambda i: (0, 0)),
            pl.BlockSpec((1, D), lambda i: (0, 0)),
            pl.BlockSpec((D, D), lambda i: (0, 0)),
        ],
        out_specs=pl.BlockSpec((BS, D), lambda i: (i, 0)),
        out_shape=jax.ShapeDtypeStruct((N, D), jnp.float32),
    )(parts, hs, degT, b, g, bb, W)


def _tclast_body(parts_ref, hs_ref, degT_ref, b_ref, out_ref):
    p = parts_ref[...]
    dinv = _dinv_of(degT_ref)
    y = dinv * (p[0, :, :DOUT] + p[1, :, :DOUT] + hs_ref[...][:, :DOUT]) + b_ref[...]
    m = jnp.max(y, axis=1, keepdims=True)
    ym = y - m
    out_ref[...] = ym - jnp.log(jnp.sum(jnp.exp(ym), axis=1, keepdims=True))


def _tclast(parts, hs, degT, b):
    return pl.pallas_call(
        _tclast_body,
        grid=(GRID,),
        in_specs=[
            pl.BlockSpec((NCORES, BS, D), lambda i: (0, i, 0)),
            pl.BlockSpec((BS, D), lambda i: (i, 0)),
            pl.BlockSpec((BS, 2), lambda i: (i, 0)),
            pl.BlockSpec((1, DOUT), lambda i: (0, 0)),
        ],
        out_specs=pl.BlockSpec((BS, DOUT), lambda i: (i, 0)),
        out_shape=jax.ShapeDtypeStruct((N, DOUT), jnp.float32),
    )(parts, hs, degT, b)


def kernel(x, edge_index, W1, b1, W2, b2, W3, b3, ln1_g, ln1_b, ln2_g, ln2_b):
    ei_pad = edge_index

    parts_deg = _deg_call(ei_pad)
    degT = parts_deg[:, :N].T

    hs1 = _tc1(x, W1, degT)
    parts1 = _agg128(hs1, ei_pad)
    hs2 = _tcmid(parts1, hs1, degT, b1.reshape(1, D), ln1_g.reshape(1, D),
                 ln1_b.reshape(1, D), W2)
    parts2 = _agg128(hs2, ei_pad)
    W3p = jnp.pad(W3, ((0, 0), (0, D - DOUT)))
    hs3 = _tcmid(parts2, hs2, degT, b2.reshape(1, D), ln2_g.reshape(1, D),
                 ln2_b.reshape(1, D), W3p)
    parts3 = _agg128(hs3, ei_pad)
    return _tclast(parts3, hs3, degT, b3.reshape(1, DOUT))
